# Initial kernel scaffold; baseline (speedup 1.0000x reference)
import jax
import jax.numpy as jnp
from jax.experimental import pallas as pl


def kernel(x, edge_index, batch, gcn_W, gcn_b, bn_g, bn_b, mlp_W1, mlp_b1, ln_g, ln_b, mlp_W2, mlp_b2):
    raise NotImplementedError("write your pallas kernel here")


# speedup vs baseline: 8.6459x; 8.6459x over previous
import functools

import jax
import jax.numpy as jnp
from jax import lax
from jax.experimental import pallas as pl
from jax.experimental.pallas import tpu as pltpu
from jax.experimental.pallas import tpu_sc as plsc

N = 10000
E = 320000
C = 128
L = 4
NPAD = 10240
K = 128
EPAD = 79 * 4096
NCH = EPAD // (32 * K)
ROWS = NPAD // 16

_mesh = plsc.VectorSubcoreMesh(core_axis_name="c", subcore_axis_name="s")


@functools.partial(
    pl.kernel,
    out_type=jax.ShapeDtypeStruct((2, NPAD, 16), jnp.float32),
    mesh=_mesh,
    scratch_types=[
        pltpu.VMEM((NCH, K), jnp.int32),
        pltpu.VMEM((K, 16), jnp.float32),
        pltpu.VMEM_SHARED((NPAD, 16), jnp.float32),
        pltpu.SemaphoreType.DMA,
    ],
)
def _deg_kernel(dst_hbm, ones_hbm, out_hbm, idx_v, ones_v, acc_sh, sem):
    c = lax.axis_index("c")
    s = lax.axis_index("s")
    wid = c * 16 + s
    pltpu.sync_copy(ones_hbm.at[pl.ds(0, K)], ones_v)
    pltpu.sync_copy(dst_hbm.at[wid], idx_v)
    pltpu.sync_copy(ones_hbm.at[pl.ds(s * ROWS, ROWS)],
                    acc_sh.at[pl.ds(s * ROWS, ROWS)])
    plsc.subcore_barrier()

    def body(j, carry):
        pltpu.sync_copy(ones_v, acc_sh.at[idx_v.at[j]], add=True)
        return carry

    lax.fori_loop(0, NCH, body, 0)
    plsc.subcore_barrier()
    pltpu.sync_copy(acc_sh.at[pl.ds(s * ROWS, ROWS)],
                    out_hbm.at[c, pl.ds(s * ROWS, ROWS)])


@functools.partial(
    pl.kernel,
    out_type=jax.ShapeDtypeStruct((2, NPAD, C), jnp.float32),
    mesh=_mesh,
    scratch_types=[
        pltpu.VMEM((NCH, K), jnp.int32),
        pltpu.VMEM((NCH, K), jnp.int32),
        pltpu.VMEM((K, C), jnp.float32),
        pltpu.VMEM_SHARED((NPAD, C), jnp.float32),
        pltpu.SemaphoreType.DMA,
    ],
)
def _agg_kernel(hwn_hbm, src_hbm, dst_hbm, out_hbm,
                sidx_v, didx_v, rows_v, acc_sh, sem):
    c = lax.axis_index("c")
    s = lax.axis_index("s")
    wid = c * 16 + s
    pltpu.sync_copy(src_hbm.at[wid], sidx_v)
    pltpu.sync_copy(dst_hbm.at[wid], didx_v)
    pltpu.sync_copy(hwn_hbm.at[pl.ds(s * ROWS, ROWS)],
                    acc_sh.at[pl.ds(s * ROWS, ROWS)])
    plsc.subcore_barrier()

    def body(j, carry):
        pltpu.async_copy(hwn_hbm.at[sidx_v.at[j]], rows_v, sem).wait()
        pltpu.sync_copy(rows_v, acc_sh.at[didx_v.at[j]], add=True)
        return carry

    lax.fori_loop(0, NCH, body, 0)
    plsc.subcore_barrier()
    pltpu.sync_copy(acc_sh.at[pl.ds(s * ROWS, ROWS)],
                    out_hbm.at[c, pl.ds(s * ROWS, ROWS)])


BLK = 512
GRID = NPAD // BLK


def _pre_body(h_ref, w_ref, deg_ref, out_ref):
    dinv = lax.rsqrt(deg_ref[...])
    hw = jnp.dot(h_ref[...], w_ref[...], preferred_element_type=jnp.float32)
    out_ref[...] = hw * dinv


def _pre_call(h, w, deg):
    return pl.pallas_call(
        _pre_body,
        grid=(GRID,),
        in_specs=[
            pl.BlockSpec((BLK, C), lambda i: (i, 0)),
            pl.BlockSpec((C, C), lambda i: (0, 0)),
            pl.BlockSpec((BLK, 1), lambda i: (i, 0)),
        ],
        out_specs=pl.BlockSpec((BLK, C), lambda i: (i, 0)),
        out_shape=jax.ShapeDtypeStruct((NPAD, C), jnp.float32),
    )(h, w, deg)


def _post_body(acc_ref, hwn_ref, deg_ref, gcnb_ref, bng_ref, bnb_ref,
               w1_ref, b1_ref, lng_ref, lnb_ref, w2_ref, b2_ref,
               h_out, t_out, sum_ref, sq_ref):
    p = pl.program_id(0)
    i = pl.program_id(1)
    accblk = acc_ref[0] + acc_ref[1] - hwn_ref[...]
    dinv = lax.rsqrt(deg_ref[...])
    agg = accblk * dinv + gcnb_ref[...]
    rows = i * BLK + lax.broadcasted_iota(jnp.int32, (BLK, 1), 0)
    mask = rows < N

    @pl.when(p == 0)
    def _():
        m = jnp.where(mask, agg, 0.0)
        ssum = jnp.sum(m, axis=0, keepdims=True)
        ssq = jnp.sum(m * m, axis=0, keepdims=True)

        @pl.when(i == 0)
        def _():
            sum_ref[...] = ssum
            sq_ref[...] = ssq

        @pl.when(i > 0)
        def _():
            sum_ref[...] += ssum
            sq_ref[...] += ssq

    @pl.when(p == 1)
    def _():
        inv_n = jnp.float32(1.0 / N)
        mean = sum_ref[...] * inv_n
        var = sq_ref[...] * inv_n - mean * mean
        rstd = lax.rsqrt(var + 1e-5)
        hbn = (agg - mean) * rstd * bng_ref[...] + bnb_ref[...]
        h2 = jnp.maximum(hbn, 0.0)
        h_out[...] = h2
        tt = jnp.dot(h2, w1_ref[...], preferred_element_type=jnp.float32)
        tt = tt + b1_ref[...]
        mu = jnp.mean(tt, axis=1, keepdims=True)
        vv = jnp.mean(tt * tt, axis=1, keepdims=True) - mu * mu
        tt = (tt - mu) * lax.rsqrt(vv + 1e-5) * lng_ref[...] + lnb_ref[...]
        tt = jnp.maximum(tt, 0.0)
        t_out[...] = jnp.dot(tt, w2_ref[...],
                             preferred_element_type=jnp.float32) + b2_ref[...]


def _post_call(acc, hwn, deg, gcnb, bng, bnb, w1, b1, lng, lnb, w2, b2):
    vec = pl.BlockSpec((1, C), lambda p, i: (0, 0))
    mat = pl.BlockSpec((C, C), lambda p, i: (0, 0))
    return pl.pallas_call(
        _post_body,
        grid=(2, GRID),
        in_specs=[
            pl.BlockSpec((2, BLK, C), lambda p, i: (0, i, 0)),
            pl.BlockSpec((BLK, C), lambda p, i: (i, 0)),
            pl.BlockSpec((BLK, 1), lambda p, i: (i, 0)),
            vec, vec, vec, mat, vec, vec, vec, mat, vec,
        ],
        out_specs=[
            pl.BlockSpec((BLK, C), lambda p, i: (i, 0)),
            pl.BlockSpec((BLK, C), lambda p, i: (i, 0)),
        ],
        out_shape=[
            jax.ShapeDtypeStruct((NPAD, C), jnp.float32),
            jax.ShapeDtypeStruct((NPAD, C), jnp.float32),
        ],
        scratch_shapes=[
            pltpu.VMEM((1, C), jnp.float32),
            pltpu.VMEM((1, C), jnp.float32),
        ],
    )(acc, hwn, deg, gcnb, bng, bnb, w1, b1, lng, lnb, w2, b2)


def kernel(x, edge_index, batch, gcn_W, gcn_b, bn_g, bn_b,
           mlp_W1, mlp_b1, ln_g, ln_b, mlp_W2, mlp_b2):
    src = edge_index[0]
    dst = edge_index[1]
    pad = EPAD - E
    src_p = jnp.concatenate([src, jnp.full((pad,), N, jnp.int32)])
    dst_p = jnp.concatenate([dst, jnp.full((pad,), N, jnp.int32)])
    src_r = src_p.reshape(32, NCH, K)
    dst_r = dst_p.reshape(32, NCH, K)
    ones_nd = jnp.ones((NPAD, 16), jnp.float32)

    deg_parts = _deg_kernel(dst_r, ones_nd)
    deg = (deg_parts[0, :, 0] + deg_parts[1, :, 0] - 1.0).reshape(NPAD, 1)

    x_pad = jnp.zeros((NPAD, C), jnp.float32).at[:N].set(x)

    def step(h, ws):
        w, gb, bg, bb, w1, b1, lg, lb, w2, b2 = ws
        hwn = _pre_call(h, w, deg)
        acc = _agg_kernel(hwn, src_r, dst_r)
        h2, t = _post_call(acc, hwn, deg, gb, bg, bb, w1, b1, lg, lb, w2, b2)
        return h2, t

    r = lambda a: a.reshape(L, 1, C)
    _, ts = lax.scan(step, x_pad,
                     (gcn_W, r(gcn_b), r(bn_g), r(bn_b), mlp_W1, r(mlp_b1),
                      r(ln_g), r(ln_b), mlp_W2, r(mlp_b2)))
    return ts[:, :N, :].transpose(1, 0, 2).reshape(1, N, L * C)


# speedup vs baseline: 18.1374x; 2.0978x over previous
import functools

import jax
import jax.numpy as jnp
from jax import lax
from jax.experimental import pallas as pl
from jax.experimental.pallas import tpu as pltpu
from jax.experimental.pallas import tpu_sc as plsc

N = 10000
E = 320000
C = 128
L = 4
NPAD = 10240
EPAD = 79 * 4096
K = 128
NCH = EPAD // (32 * K)
ROWS = NPAD // 16

_mesh = plsc.VectorSubcoreMesh(core_axis_name="c", subcore_axis_name="s")


@functools.partial(
    pl.kernel,
    out_type=jax.ShapeDtypeStruct((2, NPAD, 16), jnp.float32),
    mesh=_mesh,
    scratch_types=[
        pltpu.VMEM((NCH, K), jnp.int32),
        pltpu.VMEM((K, 16), jnp.float32),
        pltpu.VMEM_SHARED((NPAD, 16), jnp.float32),
        pltpu.SemaphoreType.DMA,
    ],
)
def _deg_kernel(dst_hbm, ones_hbm, out_hbm, idx_v, ones_v, acc_sh, sem):
    c = lax.axis_index("c")
    s = lax.axis_index("s")
    wid = c * 16 + s
    pltpu.sync_copy(ones_hbm.at[pl.ds(0, K)], ones_v)
    pltpu.sync_copy(dst_hbm.at[wid], idx_v)
    pltpu.sync_copy(ones_hbm.at[pl.ds(s * ROWS, ROWS)],
                    acc_sh.at[pl.ds(s * ROWS, ROWS)])
    plsc.subcore_barrier()

    def fire(j, carry):
        pltpu.async_copy(ones_v, acc_sh.at[idx_v.at[j]], sem, add=True)
        return carry

    lax.fori_loop(0, NCH, fire, 0)

    def drain(j, carry):
        pltpu.make_async_copy(ones_v, acc_sh.at[idx_v.at[0]], sem).wait()
        return carry

    lax.fori_loop(0, NCH, drain, 0)
    plsc.subcore_barrier()
    pltpu.sync_copy(acc_sh.at[pl.ds(s * ROWS, ROWS)],
                    out_hbm.at[c, pl.ds(s * ROWS, ROWS)])


@functools.partial(
    pl.kernel,
    out_type=jax.ShapeDtypeStruct((2, NPAD, C), jnp.float32),
    mesh=_mesh,
    scratch_types=[
        pltpu.VMEM((NCH, K), jnp.int32),
        pltpu.VMEM((2, 2, K), jnp.int32),
        pltpu.VMEM((2, K, C), jnp.float32),
        pltpu.VMEM_SHARED((NPAD, C), jnp.float32),
        pltpu.SemaphoreType.DMA,
        pltpu.SemaphoreType.DMA,
    ],
)
def _agg_kernel(hwn_hbm, eidx_hbm, out_hbm,
                pidx_v, idx_ring, rows_v, acc_sh, gsem, ssem):
    c = lax.axis_index("c")
    s = lax.axis_index("s")
    wid = c * 16 + s
    pltpu.sync_copy(eidx_hbm.at[wid], pidx_v)
    pltpu.sync_copy(hwn_hbm.at[pl.ds(s * ROWS, ROWS)],
                    acc_sh.at[pl.ds(s * ROWS, ROWS)])

    def unpack(j, slot):
        for k in range(K // 16):
            pv = pidx_v[j, pl.ds(k * 16, 16)]
            idx_ring[slot, 0, pl.ds(k * 16, 16)] = lax.shift_right_logical(
                pv, 16)
            idx_ring[slot, 1, pl.ds(k * 16, 16)] = lax.bitwise_and(
                pv, 0xFFFF)

    unpack(0, 0)
    pltpu.async_copy(hwn_hbm.at[idx_ring.at[0, 0]], rows_v.at[0], gsem)
    plsc.subcore_barrier()

    def body(j, carry):
        b = lax.rem(j, 2)
        bn = 1 - b
        pltpu.make_async_copy(hwn_hbm.at[idx_ring.at[b, 0]],
                              rows_v.at[b], gsem).wait()

        @pl.when(j > 0)
        def _():
            pltpu.make_async_copy(rows_v.at[bn],
                                  acc_sh.at[idx_ring.at[bn, 1]], ssem).wait()

        pltpu.async_copy(rows_v.at[b], acc_sh.at[idx_ring.at[b, 1]],
                         ssem, add=True)

        @pl.when(j + 1 < NCH)
        def _():
            unpack(j + 1, bn)
            pltpu.async_copy(hwn_hbm.at[idx_ring.at[bn, 0]],
                             rows_v.at[bn], gsem)

        return carry

    lax.fori_loop(0, NCH, body, 0)
    pltpu.make_async_copy(rows_v.at[lax.rem(NCH - 1, 2)],
                          acc_sh.at[idx_ring.at[lax.rem(NCH - 1, 2), 1]],
                          ssem).wait()
    plsc.subcore_barrier()
    pltpu.sync_copy(acc_sh.at[pl.ds(s * ROWS, ROWS)],
                    out_hbm.at[c, pl.ds(s * ROWS, ROWS)])


BLK = 512
GRID = NPAD // BLK


def _pre_body(h_ref, w_ref, deg_ref, out_ref):
    dinv = lax.rsqrt(deg_ref[...])
    hw = jnp.dot(h_ref[...], w_ref[...], preferred_element_type=jnp.float32)
    out_ref[...] = hw * dinv


def _pre_call(h, w, deg):
    return pl.pallas_call(
        _pre_body,
        grid=(GRID,),
        in_specs=[
            pl.BlockSpec((BLK, C), lambda i: (i, 0)),
            pl.BlockSpec((C, C), lambda i: (0, 0)),
            pl.BlockSpec((BLK, 1), lambda i: (i, 0)),
        ],
        out_specs=pl.BlockSpec((BLK, C), lambda i: (i, 0)),
        out_shape=jax.ShapeDtypeStruct((NPAD, C), jnp.float32),
    )(h, w, deg)


def _agg_to_bn(acc_ref, hwn_ref, deg_ref, gcnb_ref):
    accblk = acc_ref[0] + acc_ref[1] - hwn_ref[...]
    dinv = lax.rsqrt(deg_ref[...])
    return accblk * dinv + gcnb_ref[...]


def _postbn_body(acc_ref, hwn_ref, deg_ref, gcnb_ref, bng_ref, bnb_ref,
                 wn_ref, h_out, hwnn_out, sum_ref, sq_ref):
    p = pl.program_id(0)
    i = pl.program_id(1)
    agg = _agg_to_bn(acc_ref, hwn_ref, deg_ref, gcnb_ref)

    @pl.when(p == 0)
    def _():
        rows = i * BLK + lax.broadcasted_iota(jnp.int32, (BLK, 1), 0)
        m = jnp.where(rows < N, agg, 0.0)
        ssum = jnp.sum(m, axis=0, keepdims=True)
        ssq = jnp.sum(m * m, axis=0, keepdims=True)

        @pl.when(i == 0)
        def _():
            sum_ref[...] = ssum
            sq_ref[...] = ssq

        @pl.when(i > 0)
        def _():
            sum_ref[...] += ssum
            sq_ref[...] += ssq

    @pl.when(p == 1)
    def _():
        inv_n = jnp.float32(1.0 / N)
        mean = sum_ref[...] * inv_n
        var = sq_ref[...] * inv_n - mean * mean
        rstd = lax.rsqrt(var + 1e-5)
        h2 = jnp.maximum((agg - mean) * rstd * bng_ref[...] + bnb_ref[...],
                         0.0)
        h_out[...] = h2
        dinv = lax.rsqrt(deg_ref[...])
        hwnn_out[...] = dinv * jnp.dot(h2, wn_ref[...],
                                       preferred_element_type=jnp.float32)


def _postbn_call(acc, hwn, deg, gcnb, bng, bnb, wnext):
    vec = pl.BlockSpec((1, C), lambda p, i: (0, 0))
    ospec = pl.BlockSpec((BLK, C), lambda p, i: (i * p, 0))
    return pl.pallas_call(
        _postbn_body,
        grid=(2, GRID),
        in_specs=[
            pl.BlockSpec((2, BLK, C), lambda p, i: (0, i, 0)),
            pl.BlockSpec((BLK, C), lambda p, i: (i, 0)),
            pl.BlockSpec((BLK, 1), lambda p, i: (i, 0)),
            vec, vec, vec,
            pl.BlockSpec((C, C), lambda p, i: (0, 0)),
        ],
        out_specs=[ospec, ospec],
        out_shape=[
            jax.ShapeDtypeStruct((NPAD, C), jnp.float32),
            jax.ShapeDtypeStruct((NPAD, C), jnp.float32),
        ],
        scratch_shapes=[
            pltpu.VMEM((1, C), jnp.float32),
            pltpu.VMEM((1, C), jnp.float32),
        ],
    )(acc, hwn, deg, gcnb, bng, bnb, wnext)


def _mlp_body(h_ref, w1_ref, b1_ref, lng_ref, lnb_ref, w2_ref, b2_ref,
              t_out):
    tt = jnp.dot(h_ref[...], w1_ref[...], preferred_element_type=jnp.float32)
    tt = tt + b1_ref[...]
    mu = jnp.mean(tt, axis=1, keepdims=True)
    vv = jnp.mean(tt * tt, axis=1, keepdims=True) - mu * mu
    tt = (tt - mu) * lax.rsqrt(vv + 1e-5) * lng_ref[...] + lnb_ref[...]
    tt = jnp.maximum(tt, 0.0)
    t_out[...] = jnp.dot(tt, w2_ref[...],
                         preferred_element_type=jnp.float32) + b2_ref[...]


def _mlp_call(h2, w1, b1, lng, lnb, w2, b2):
    vec = pl.BlockSpec((1, C), lambda i: (0, 0))
    mat = pl.BlockSpec((C, C), lambda i: (0, 0))
    return pl.pallas_call(
        _mlp_body,
        grid=(GRID,),
        in_specs=[pl.BlockSpec((BLK, C), lambda i: (i, 0)),
                  mat, vec, vec, vec, mat, vec],
        out_specs=pl.BlockSpec((BLK, C), lambda i: (i, 0)),
        out_shape=jax.ShapeDtypeStruct((NPAD, C), jnp.float32),
    )(h2, w1, b1, lng, lnb, w2, b2)


def _mm_body(h_ref, w_ref, out_ref):
    out_ref[...] = jnp.dot(h_ref[...], w_ref[...],
                           preferred_element_type=jnp.float32)


def _mm_call(h, w):
    return pl.pallas_call(
        _mm_body,
        grid=(GRID,),
        in_specs=[
            pl.BlockSpec((BLK, C), lambda i: (i, 0)),
            pl.BlockSpec((C, C), lambda i: (0, 0)),
        ],
        out_specs=pl.BlockSpec((BLK, C), lambda i: (i, 0)),
        out_shape=jax.ShapeDtypeStruct((NPAD, C), jnp.float32),
    )(h, w)


def _scale_body(xw_ref, deg_ref, out_ref):
    out_ref[...] = xw_ref[...] * lax.rsqrt(deg_ref[...])


def _scale_call(xw, deg):
    return pl.pallas_call(
        _scale_body,
        grid=(GRID,),
        in_specs=[
            pl.BlockSpec((BLK, C), lambda i: (i, 0)),
            pl.BlockSpec((BLK, 1), lambda i: (i, 0)),
        ],
        out_specs=pl.BlockSpec((BLK, C), lambda i: (i, 0)),
        out_shape=jax.ShapeDtypeStruct((NPAD, C), jnp.float32),
    )(xw, deg)


PBLK = 400
PGRID = N // PBLK


def _pack_body(t0_ref, t1_ref, t2_ref, t3_ref, out_ref):
    out_ref[0, :, 0 * C:1 * C] = t0_ref[...]
    out_ref[0, :, 1 * C:2 * C] = t1_ref[...]
    out_ref[0, :, 2 * C:3 * C] = t2_ref[...]
    out_ref[0, :, 3 * C:4 * C] = t3_ref[...]


def _pack_call(ts):
    tspec = pl.BlockSpec((PBLK, C), lambda n: (n, 0))
    return pl.pallas_call(
        _pack_body,
        grid=(PGRID,),
        in_specs=[tspec, tspec, tspec, tspec],
        out_specs=pl.BlockSpec((1, PBLK, L * C), lambda n: (0, n, 0)),
        out_shape=jax.ShapeDtypeStruct((1, N, L * C), jnp.float32),
    )(*ts)


def kernel(x, edge_index, batch, gcn_W, gcn_b, bn_g, bn_b,
           mlp_W1, mlp_b1, ln_g, ln_b, mlp_W2, mlp_b2):
    src = edge_index[0]
    dst = edge_index[1]
    pad = EPAD - E
    trash = N + (jnp.arange(pad, dtype=jnp.int32) % (NPAD - N))
    src_p = jnp.concatenate([src, trash])
    dst_p = jnp.concatenate([dst, trash])
    packed = jnp.left_shift(src_p, 16) + dst_p
    eidx = packed.reshape(32, NCH, K)
    dst_d = dst_p.reshape(32, NCH, K)
    ones_nd = jnp.ones((NPAD, 16), jnp.float32)

    x_pad = jnp.zeros((NPAD, C), jnp.float32).at[:N].set(x)

    deg_parts = _deg_kernel(dst_d, ones_nd)
    xw = _mm_call(x_pad, gcn_W[0])
    deg = (deg_parts[0, :, 0] + deg_parts[1, :, 0] - 1.0).reshape(NPAD, 1)
    hwn = _scale_call(xw, deg)

    r = lambda a: a.reshape(1, C)
    ts = []
    for i in range(L):
        acc = _agg_kernel(hwn, eidx)
        h, hwn = _postbn_call(acc, hwn, deg, r(gcn_b[i]),
                              r(bn_g[i]), r(bn_b[i]), gcn_W[(i + 1) % L])
        t = _mlp_call(h, mlp_W1[i], r(mlp_b1[i]), r(ln_g[i]), r(ln_b[i]),
                      mlp_W2[i], r(mlp_b2[i]))
        ts.append(t)
    return _pack_call(ts)


# speedup vs baseline: 19.8249x; 1.0930x over previous
import functools

import jax
import jax.numpy as jnp
from jax import lax
from jax.experimental import pallas as pl
from jax.experimental.pallas import tpu as pltpu
from jax.experimental.pallas import tpu_sc as plsc

N = 10000
E = 320000
C = 128
L = 4
NPAD = 10240
EPAD = 79 * 4096
K = 128
NCH = EPAD // (32 * K)
ROWS = NPAD // 16

_mesh = plsc.VectorSubcoreMesh(core_axis_name="c", subcore_axis_name="s")


@functools.partial(
    pl.kernel,
    out_type=jax.ShapeDtypeStruct((2, NPAD, 16), jnp.float32),
    mesh=_mesh,
    scratch_types=[
        pltpu.VMEM((NCH, K), jnp.int32),
        pltpu.VMEM((K, 16), jnp.float32),
        pltpu.VMEM_SHARED((NPAD, 16), jnp.float32),
        pltpu.SemaphoreType.DMA,
    ],
)
def _deg_kernel(dst_hbm, ones_hbm, out_hbm, idx_v, ones_v, acc_sh, sem):
    c = lax.axis_index("c")
    s = lax.axis_index("s")
    wid = c * 16 + s
    pltpu.sync_copy(ones_hbm.at[pl.ds(0, K)], ones_v)
    pltpu.sync_copy(dst_hbm.at[wid], idx_v)
    pltpu.sync_copy(ones_hbm.at[pl.ds(s * ROWS, ROWS)],
                    acc_sh.at[pl.ds(s * ROWS, ROWS)])
    plsc.subcore_barrier()

    def fire(j, carry):
        pltpu.async_copy(ones_v, acc_sh.at[idx_v.at[j]], sem, add=True)
        return carry

    lax.fori_loop(0, NCH, fire, 0)

    def drain(j, carry):
        pltpu.make_async_copy(ones_v, acc_sh.at[idx_v.at[0]], sem).wait()
        return carry

    lax.fori_loop(0, NCH, drain, 0)
    plsc.subcore_barrier()
    pltpu.sync_copy(acc_sh.at[pl.ds(s * ROWS, ROWS)],
                    out_hbm.at[c, pl.ds(s * ROWS, ROWS)])


@functools.partial(
    pl.kernel,
    out_type=jax.ShapeDtypeStruct((2, NPAD, C), jnp.float32),
    mesh=_mesh,
    scratch_types=[
        pltpu.VMEM((NCH, K), jnp.int32),
        pltpu.VMEM((2, 2, K), jnp.int32),
        pltpu.VMEM((2, K, C), jnp.float32),
        pltpu.VMEM_SHARED((NPAD, C), jnp.float32),
        pltpu.SemaphoreType.DMA,
        pltpu.SemaphoreType.DMA,
    ],
)
def _agg_kernel(hwn_hbm, eidx_hbm, out_hbm,
                pidx_v, idx_ring, rows_v, acc_sh, gsem, ssem):
    c = lax.axis_index("c")
    s = lax.axis_index("s")
    wid = c * 16 + s
    pltpu.sync_copy(eidx_hbm.at[wid], pidx_v)
    pltpu.sync_copy(hwn_hbm.at[pl.ds(s * ROWS, ROWS)],
                    acc_sh.at[pl.ds(s * ROWS, ROWS)])

    def unpack(j, slot):
        for k in range(K // 16):
            pv = pidx_v[j, pl.ds(k * 16, 16)]
            idx_ring[slot, 0, pl.ds(k * 16, 16)] = lax.shift_right_logical(
                pv, 16)
            idx_ring[slot, 1, pl.ds(k * 16, 16)] = lax.bitwise_and(
                pv, 0xFFFF)

    unpack(0, 0)
    pltpu.async_copy(hwn_hbm.at[idx_ring.at[0, 0]], rows_v.at[0], gsem)
    plsc.subcore_barrier()

    def body(j, carry):
        b = lax.rem(j, 2)
        bn = 1 - b
        pltpu.make_async_copy(hwn_hbm.at[idx_ring.at[b, 0]],
                              rows_v.at[b], gsem).wait()

        @pl.when(j > 0)
        def _():
            pltpu.make_async_copy(rows_v.at[bn],
                                  acc_sh.at[idx_ring.at[bn, 1]], ssem).wait()

        pltpu.async_copy(rows_v.at[b], acc_sh.at[idx_ring.at[b, 1]],
                         ssem, add=True)

        @pl.when(j + 1 < NCH)
        def _():
            unpack(j + 1, bn)
            pltpu.async_copy(hwn_hbm.at[idx_ring.at[bn, 0]],
                             rows_v.at[bn], gsem)

        return carry

    lax.fori_loop(0, NCH, body, 0)
    pltpu.make_async_copy(rows_v.at[lax.rem(NCH - 1, 2)],
                          acc_sh.at[idx_ring.at[lax.rem(NCH - 1, 2), 1]],
                          ssem).wait()
    plsc.subcore_barrier()
    pltpu.sync_copy(acc_sh.at[pl.ds(s * ROWS, ROWS)],
                    out_hbm.at[c, pl.ds(s * ROWS, ROWS)])


BLK = 1024
GRID = NPAD // BLK


def _pre_body(h_ref, w_ref, deg_ref, out_ref):
    dinv = lax.rsqrt(deg_ref[...])
    hw = jnp.dot(h_ref[...], w_ref[...], preferred_element_type=jnp.float32)
    out_ref[...] = hw * dinv


def _pre_call(h, w, deg):
    return pl.pallas_call(
        _pre_body,
        grid=(GRID,),
        in_specs=[
            pl.BlockSpec((BLK, C), lambda i: (i, 0)),
            pl.BlockSpec((C, C), lambda i: (0, 0)),
            pl.BlockSpec((BLK, 1), lambda i: (i, 0)),
        ],
        out_specs=pl.BlockSpec((BLK, C), lambda i: (i, 0)),
        out_shape=jax.ShapeDtypeStruct((NPAD, C), jnp.float32),
    )(h, w, deg)


def _agg_to_bn(acc_ref, hwn_ref, deg_ref, gcnb_ref):
    accblk = acc_ref[0] + acc_ref[1] - hwn_ref[...]
    dinv = lax.rsqrt(deg_ref[...])
    return accblk * dinv + gcnb_ref[...]


def _postbn_body(acc_ref, hwn_ref, deg_ref, gcnb_ref, bng_ref, bnb_ref,
                 wn_ref, h_out, hwnn_out, sum_ref, sq_ref):
    p = pl.program_id(0)
    i = pl.program_id(1)
    agg = _agg_to_bn(acc_ref, hwn_ref, deg_ref, gcnb_ref)

    @pl.when(p == 0)
    def _():
        rows = i * BLK + lax.broadcasted_iota(jnp.int32, (BLK, 1), 0)
        m = jnp.where(rows < N, agg, 0.0)
        ssum = jnp.sum(m, axis=0, keepdims=True)
        ssq = jnp.sum(m * m, axis=0, keepdims=True)

        @pl.when(i == 0)
        def _():
            sum_ref[...] = ssum
            sq_ref[...] = ssq

        @pl.when(i > 0)
        def _():
            sum_ref[...] += ssum
            sq_ref[...] += ssq

    @pl.when(p == 1)
    def _():
        inv_n = jnp.float32(1.0 / N)
        mean = sum_ref[...] * inv_n
        var = sq_ref[...] * inv_n - mean * mean
        rstd = lax.rsqrt(var + 1e-5)
        h2 = jnp.maximum((agg - mean) * rstd * bng_ref[...] + bnb_ref[...],
                         0.0)
        h_out[...] = h2
        dinv = lax.rsqrt(deg_ref[...])
        hwnn_out[...] = dinv * jnp.dot(h2, wn_ref[...],
                                       preferred_element_type=jnp.float32)


def _postbn_call(acc, hwn, deg, gcnb, bng, bnb, wnext):
    vec = pl.BlockSpec((1, C), lambda p, i: (0, 0))
    ospec = pl.BlockSpec((BLK, C), lambda p, i: (i * p, 0))
    return pl.pallas_call(
        _postbn_body,
        grid=(2, GRID),
        in_specs=[
            pl.BlockSpec((2, BLK, C), lambda p, i: (0, i, 0)),
            pl.BlockSpec((BLK, C), lambda p, i: (i, 0)),
            pl.BlockSpec((BLK, 1), lambda p, i: (i, 0)),
            vec, vec, vec,
            pl.BlockSpec((C, C), lambda p, i: (0, 0)),
        ],
        out_specs=[ospec, ospec],
        out_shape=[
            jax.ShapeDtypeStruct((NPAD, C), jnp.float32),
            jax.ShapeDtypeStruct((NPAD, C), jnp.float32),
        ],
        scratch_shapes=[
            pltpu.VMEM((1, C), jnp.float32),
            pltpu.VMEM((1, C), jnp.float32),
        ],
    )(acc, hwn, deg, gcnb, bng, bnb, wnext)


MBLK = 400
MGRID = N // MBLK


def _mlp_body(out_in_ref, h_ref, w1_ref, b1_ref, lng_ref, lnb_ref,
              w2_ref, b2_ref, t_out):
    tt = jnp.dot(h_ref[...], w1_ref[...], preferred_element_type=jnp.float32)
    tt = tt + b1_ref[...]
    mu = jnp.mean(tt, axis=1, keepdims=True)
    vv = jnp.mean(tt * tt, axis=1, keepdims=True) - mu * mu
    tt = (tt - mu) * lax.rsqrt(vv + 1e-5) * lng_ref[...] + lnb_ref[...]
    tt = jnp.maximum(tt, 0.0)
    t_out[...] = jnp.dot(tt, w2_ref[...],
                         preferred_element_type=jnp.float32)[None] + b2_ref[...]


def _mlp_call(i, out_buf, h2, w1, b1, lng, lnb, w2, b2):
    vec = pl.BlockSpec((1, C), lambda n: (0, 0))
    mat = pl.BlockSpec((C, C), lambda n: (0, 0))
    return pl.pallas_call(
        _mlp_body,
        grid=(MGRID,),
        in_specs=[
            pl.BlockSpec((1, MBLK, C), lambda n: (0, n, i)),
            pl.BlockSpec((MBLK, C), lambda n: (n, 0)),
            mat, vec, vec, vec, mat, vec,
        ],
        out_specs=pl.BlockSpec((1, MBLK, C), lambda n: (0, n, i)),
        out_shape=jax.ShapeDtypeStruct((1, N, L * C), jnp.float32),
        input_output_aliases={0: 0},
    )(out_buf, h2, w1, b1, lng, lnb, w2, b2)


def _mm_body(h_ref, w_ref, out_ref):
    out_ref[...] = jnp.dot(h_ref[...], w_ref[...],
                           preferred_element_type=jnp.float32)


def _mm_call(h, w):
    return pl.pallas_call(
        _mm_body,
        grid=(GRID,),
        in_specs=[
            pl.BlockSpec((BLK, C), lambda i: (i, 0)),
            pl.BlockSpec((C, C), lambda i: (0, 0)),
        ],
        out_specs=pl.BlockSpec((BLK, C), lambda i: (i, 0)),
        out_shape=jax.ShapeDtypeStruct((NPAD, C), jnp.float32),
    )(h, w)


def _scale_body(xw_ref, deg_ref, out_ref):
    out_ref[...] = xw_ref[...] * lax.rsqrt(deg_ref[...])


def _scale_call(xw, deg):
    return pl.pallas_call(
        _scale_body,
        grid=(GRID,),
        in_specs=[
            pl.BlockSpec((BLK, C), lambda i: (i, 0)),
            pl.BlockSpec((BLK, 1), lambda i: (i, 0)),
        ],
        out_specs=pl.BlockSpec((BLK, C), lambda i: (i, 0)),
        out_shape=jax.ShapeDtypeStruct((NPAD, C), jnp.float32),
    )(xw, deg)


def kernel(x, edge_index, batch, gcn_W, gcn_b, bn_g, bn_b,
           mlp_W1, mlp_b1, ln_g, ln_b, mlp_W2, mlp_b2):
    src = edge_index[0]
    dst = edge_index[1]
    pad = EPAD - E
    trash = N + (jnp.arange(pad, dtype=jnp.int32) % (NPAD - N))
    src_p = jnp.concatenate([src, trash])
    dst_p = jnp.concatenate([dst, trash])
    packed = jnp.left_shift(src_p, 16) + dst_p
    eidx = packed.reshape(32, NCH, K)
    dst_d = dst_p.reshape(32, NCH, K)
    ones_nd = jnp.ones((NPAD, 16), jnp.float32)

    x_pad = jnp.zeros((NPAD, C), jnp.float32).at[:N].set(x)

    deg_parts = _deg_kernel(dst_d, ones_nd)
    xw = _mm_call(x_pad, gcn_W[0])
    deg = (deg_parts[0, :, 0] + deg_parts[1, :, 0] - 1.0).reshape(NPAD, 1)
    hwn = _scale_call(xw, deg)

    r = lambda a: a.reshape(1, C)
    out = jnp.zeros((1, N, L * C), jnp.float32)
    for i in range(L):
        acc = _agg_kernel(hwn, eidx)
        h, hwn = _postbn_call(acc, hwn, deg, r(gcn_b[i]),
                              r(bn_g[i]), r(bn_b[i]), gcn_W[(i + 1) % L])
        out = _mlp_call(i, out, h, mlp_W1[i], r(mlp_b1[i]), r(ln_g[i]),
                        r(ln_b[i]), mlp_W2[i], r(mlp_b2[i]))
    return out


# speedup vs baseline: 19.8721x; 1.0024x over previous
import functools

import jax
import jax.numpy as jnp
from jax import lax
from jax.experimental import pallas as pl
from jax.experimental.pallas import tpu as pltpu
from jax.experimental.pallas import tpu_sc as plsc

N = 10000
E = 320000
C = 128
L = 4
NPAD = 10240
EPAD = 79 * 4096
K = 128
NCH = EPAD // (32 * K)
ROWS = NPAD // 16

_mesh = plsc.VectorSubcoreMesh(core_axis_name="c", subcore_axis_name="s")


@functools.partial(
    pl.kernel,
    out_type=jax.ShapeDtypeStruct((2, NPAD, 16), jnp.float32),
    mesh=_mesh,
    scratch_types=[
        pltpu.VMEM((NCH, K), jnp.int32),
        pltpu.VMEM((K, 16), jnp.float32),
        pltpu.VMEM_SHARED((NPAD, 16), jnp.float32),
        pltpu.SemaphoreType.DMA,
    ],
)
def _deg_kernel(dst_hbm, ones_hbm, out_hbm, idx_v, ones_v, acc_sh, sem):
    c = lax.axis_index("c")
    s = lax.axis_index("s")
    wid = c * 16 + s
    pltpu.sync_copy(ones_hbm.at[pl.ds(0, K)], ones_v)
    pltpu.sync_copy(dst_hbm.at[wid], idx_v)
    pltpu.sync_copy(ones_hbm.at[pl.ds(s * ROWS, ROWS)],
                    acc_sh.at[pl.ds(s * ROWS, ROWS)])
    plsc.subcore_barrier()

    def fire(j, carry):
        pltpu.async_copy(ones_v, acc_sh.at[idx_v.at[j]], sem, add=True)
        return carry

    lax.fori_loop(0, NCH, fire, 0)

    def drain(j, carry):
        pltpu.make_async_copy(ones_v, acc_sh.at[idx_v.at[0]], sem).wait()
        return carry

    lax.fori_loop(0, NCH, drain, 0)
    plsc.subcore_barrier()
    pltpu.sync_copy(acc_sh.at[pl.ds(s * ROWS, ROWS)],
                    out_hbm.at[c, pl.ds(s * ROWS, ROWS)])


@functools.partial(
    pl.kernel,
    out_type=jax.ShapeDtypeStruct((2, NPAD, C), jnp.float32),
    mesh=_mesh,
    scratch_types=[
        pltpu.VMEM((NCH, K), jnp.int32),
        pltpu.VMEM((2, 2, K), jnp.int32),
        pltpu.VMEM((2, K, C), jnp.float32),
        pltpu.VMEM_SHARED((NPAD, C), jnp.float32),
        pltpu.SemaphoreType.DMA,
        pltpu.SemaphoreType.DMA,
    ],
)
def _agg_kernel(hwn_hbm, eidx_hbm, out_hbm,
                pidx_v, idx_ring, rows_v, acc_sh, gsem, ssem):
    c = lax.axis_index("c")
    s = lax.axis_index("s")
    wid = c * 16 + s
    pltpu.sync_copy(eidx_hbm.at[wid], pidx_v)
    pltpu.sync_copy(hwn_hbm.at[pl.ds(s * ROWS, ROWS)],
                    acc_sh.at[pl.ds(s * ROWS, ROWS)])

    def unpack(j, slot):
        for k in range(K // 16):
            pv = pidx_v[j, pl.ds(k * 16, 16)]
            idx_ring[slot, 0, pl.ds(k * 16, 16)] = lax.shift_right_logical(
                pv, 16)
            idx_ring[slot, 1, pl.ds(k * 16, 16)] = lax.bitwise_and(
                pv, 0xFFFF)

    unpack(0, 0)
    pltpu.async_copy(hwn_hbm.at[idx_ring.at[0, 0]], rows_v.at[0], gsem)
    plsc.subcore_barrier()

    def body(j, carry):
        b = lax.rem(j, 2)
        bn = 1 - b
        pltpu.make_async_copy(hwn_hbm.at[idx_ring.at[b, 0]],
                              rows_v.at[b], gsem).wait()

        @pl.when(j > 0)
        def _():
            pltpu.make_async_copy(rows_v.at[bn],
                                  acc_sh.at[idx_ring.at[bn, 1]], ssem).wait()

        pltpu.async_copy(rows_v.at[b], acc_sh.at[idx_ring.at[b, 1]],
                         ssem, add=True)

        @pl.when(j + 1 < NCH)
        def _():
            unpack(j + 1, bn)
            pltpu.async_copy(hwn_hbm.at[idx_ring.at[bn, 0]],
                             rows_v.at[bn], gsem)

        return carry

    lax.fori_loop(0, NCH, body, 0)
    pltpu.make_async_copy(rows_v.at[lax.rem(NCH - 1, 2)],
                          acc_sh.at[idx_ring.at[lax.rem(NCH - 1, 2), 1]],
                          ssem).wait()
    plsc.subcore_barrier()
    pltpu.sync_copy(acc_sh.at[pl.ds(s * ROWS, ROWS)],
                    out_hbm.at[c, pl.ds(s * ROWS, ROWS)])


BLK = 1024
GRID = NPAD // BLK


def _pre_body(h_ref, w_ref, deg_ref, out_ref):
    dinv = lax.rsqrt(deg_ref[...])
    hw = jnp.dot(h_ref[...], w_ref[...], preferred_element_type=jnp.float32)
    out_ref[...] = hw * dinv


def _pre_call(h, w, deg):
    return pl.pallas_call(
        _pre_body,
        grid=(GRID,),
        in_specs=[
            pl.BlockSpec((BLK, C), lambda i: (i, 0)),
            pl.BlockSpec((C, C), lambda i: (0, 0)),
            pl.BlockSpec((BLK, 1), lambda i: (i, 0)),
        ],
        out_specs=pl.BlockSpec((BLK, C), lambda i: (i, 0)),
        out_shape=jax.ShapeDtypeStruct((NPAD, C), jnp.float32),
    )(h, w, deg)


def _agg_to_bn(acc_ref, hwn_ref, deg_ref, gcnb_ref):
    accblk = acc_ref[0] + acc_ref[1] - hwn_ref[...]
    dinv = lax.rsqrt(deg_ref[...])
    return accblk * dinv + gcnb_ref[...]


def _postbn_body(acc_ref, hwn_ref, deg_ref, gcnb_ref, bng_ref, bnb_ref,
                 wn_ref, h_out, hwnn_out, sum_ref, sq_ref):
    p = pl.program_id(0)
    i = pl.program_id(1)
    agg = _agg_to_bn(acc_ref, hwn_ref, deg_ref, gcnb_ref)

    @pl.when(p == 0)
    def _():
        rows = i * BLK + lax.broadcasted_iota(jnp.int32, (BLK, 1), 0)
        m = jnp.where(rows < N, agg, 0.0)
        ssum = jnp.sum(m, axis=0, keepdims=True)
        ssq = jnp.sum(m * m, axis=0, keepdims=True)

        @pl.when(i == 0)
        def _():
            sum_ref[...] = ssum
            sq_ref[...] = ssq

        @pl.when(i > 0)
        def _():
            sum_ref[...] += ssum
            sq_ref[...] += ssq

    @pl.when(p == 1)
    def _():
        inv_n = jnp.float32(1.0 / N)
        mean = sum_ref[...] * inv_n
        var = sq_ref[...] * inv_n - mean * mean
        rstd = lax.rsqrt(var + 1e-5)
        h2 = jnp.maximum((agg - mean) * rstd * bng_ref[...] + bnb_ref[...],
                         0.0)
        h_out[...] = h2
        dinv = lax.rsqrt(deg_ref[...])
        hwnn_out[...] = dinv * jnp.dot(h2, wn_ref[...],
                                       preferred_element_type=jnp.float32)


def _postbn_call(acc, hwn, deg, gcnb, bng, bnb, wnext):
    vec = pl.BlockSpec((1, C), lambda p, i: (0, 0))
    ospec = pl.BlockSpec((BLK, C), lambda p, i: (i * p, 0))
    return pl.pallas_call(
        _postbn_body,
        grid=(2, GRID),
        in_specs=[
            pl.BlockSpec((2, BLK, C), lambda p, i: (0, i, 0)),
            pl.BlockSpec((BLK, C), lambda p, i: (i, 0)),
            pl.BlockSpec((BLK, 1), lambda p, i: (i, 0)),
            vec, vec, vec,
            pl.BlockSpec((C, C), lambda p, i: (0, 0)),
        ],
        out_specs=[ospec, ospec],
        out_shape=[
            jax.ShapeDtypeStruct((NPAD, C), jnp.float32),
            jax.ShapeDtypeStruct((NPAD, C), jnp.float32),
        ],
        scratch_shapes=[
            pltpu.VMEM((1, C), jnp.float32),
            pltpu.VMEM((1, C), jnp.float32),
        ],
    )(acc, hwn, deg, gcnb, bng, bnb, wnext)


MBLK = 400
MGRID = N // MBLK


def _mlp_body(out_in_ref, h_ref, w1_ref, b1_ref, lng_ref, lnb_ref,
              w2_ref, b2_ref, t_out):
    tt = jnp.dot(h_ref[...], w1_ref[...], preferred_element_type=jnp.float32)
    tt = tt + b1_ref[...]
    mu = jnp.mean(tt, axis=1, keepdims=True)
    vv = jnp.mean(tt * tt, axis=1, keepdims=True) - mu * mu
    tt = (tt - mu) * lax.rsqrt(vv + 1e-5) * lng_ref[...] + lnb_ref[...]
    tt = jnp.maximum(tt, 0.0)
    t_out[...] = jnp.dot(tt, w2_ref[...],
                         preferred_element_type=jnp.float32)[None] + b2_ref[...]


def _mlp_call(i, out_buf, h2, w1, b1, lng, lnb, w2, b2):
    vec = pl.BlockSpec((1, C), lambda n: (0, 0))
    mat = pl.BlockSpec((C, C), lambda n: (0, 0))
    return pl.pallas_call(
        _mlp_body,
        grid=(MGRID,),
        in_specs=[
            pl.BlockSpec((1, MBLK, C), lambda n: (0, n, i)),
            pl.BlockSpec((MBLK, C), lambda n: (n, 0)),
            mat, vec, vec, vec, mat, vec,
        ],
        out_specs=pl.BlockSpec((1, MBLK, C), lambda n: (0, n, i)),
        out_shape=jax.ShapeDtypeStruct((1, N, L * C), jnp.float32),
        input_output_aliases={0: 0},
    )(out_buf, h2, w1, b1, lng, lnb, w2, b2)


def _mm_body(h_ref, w_ref, out_ref):
    out_ref[...] = jnp.dot(h_ref[...], w_ref[...],
                           preferred_element_type=jnp.float32)


def _mm_call(h, w):
    return pl.pallas_call(
        _mm_body,
        grid=(GRID,),
        in_specs=[
            pl.BlockSpec((BLK, C), lambda i: (i, 0)),
            pl.BlockSpec((C, C), lambda i: (0, 0)),
        ],
        out_specs=pl.BlockSpec((BLK, C), lambda i: (i, 0)),
        out_shape=jax.ShapeDtypeStruct((NPAD, C), jnp.float32),
    )(h, w)


def _scale_body(xw_ref, parts_ref, out_ref):
    deg = parts_ref[0, :, 0:1] + parts_ref[1, :, 0:1] - 1.0
    out_ref[...] = xw_ref[...] * lax.rsqrt(deg)


def _scale_call(xw, deg_parts):
    return pl.pallas_call(
        _scale_body,
        grid=(GRID,),
        in_specs=[
            pl.BlockSpec((BLK, C), lambda i: (i, 0)),
            pl.BlockSpec((2, BLK, 16), lambda i: (0, i, 0)),
        ],
        out_specs=pl.BlockSpec((BLK, C), lambda i: (i, 0)),
        out_shape=jax.ShapeDtypeStruct((NPAD, C), jnp.float32),
    )(xw, deg_parts)


def kernel(x, edge_index, batch, gcn_W, gcn_b, bn_g, bn_b,
           mlp_W1, mlp_b1, ln_g, ln_b, mlp_W2, mlp_b2):
    src = edge_index[0]
    dst = edge_index[1]
    pad = EPAD - E
    trash = N + (jnp.arange(pad, dtype=jnp.int32) % (NPAD - N))
    src_p = jnp.concatenate([src, trash])
    dst_p = jnp.concatenate([dst, trash])
    packed = jnp.left_shift(src_p, 16) + dst_p
    eidx = packed.reshape(32, NCH, K)
    dst_d = dst_p.reshape(32, NCH, K)
    ones_nd = jnp.ones((NPAD, 16), jnp.float32)

    x_pad = jnp.zeros((NPAD, C), jnp.float32).at[:N].set(x)

    deg_parts = _deg_kernel(dst_d, ones_nd)
    xw = _mm_call(x_pad, gcn_W[0])
    hwn = _scale_call(xw, deg_parts)
    deg = (deg_parts[0, :, 0] + deg_parts[1, :, 0] - 1.0).reshape(NPAD, 1)

    r = lambda a: a.reshape(1, C)
    out = jnp.zeros((1, N, L * C), jnp.float32)
    for i in range(L):
        acc = _agg_kernel(hwn, eidx)
        h, hwn = _postbn_call(acc, hwn, deg, r(gcn_b[i]),
                              r(bn_g[i]), r(bn_b[i]), gcn_W[(i + 1) % L])
        out = _mlp_call(i, out, h, mlp_W1[i], r(mlp_b1[i]), r(ln_g[i]),
                        r(ln_b[i]), mlp_W2[i], r(mlp_b2[i]))
    return out


# speedup vs baseline: 19.9186x; 1.0023x over previous
import functools

import jax
import jax.numpy as jnp
from jax import lax
from jax.experimental import pallas as pl
from jax.experimental.pallas import tpu as pltpu
from jax.experimental.pallas import tpu_sc as plsc

N = 10000
E = 320000
C = 128
L = 4
NPAD = 10240
EPAD = 79 * 4096
K = 128
NCH = EPAD // (32 * K)
ROWS = NPAD // 16

_mesh = plsc.VectorSubcoreMesh(core_axis_name="c", subcore_axis_name="s")


@functools.partial(
    pl.kernel,
    out_type=jax.ShapeDtypeStruct((2, NPAD, 16), jnp.float32),
    mesh=_mesh,
    scratch_types=[
        pltpu.VMEM((NCH, K), jnp.int32),
        pltpu.VMEM((K, 16), jnp.float32),
        pltpu.VMEM_SHARED((NPAD, 16), jnp.float32),
        pltpu.SemaphoreType.DMA,
    ],
)
def _deg_kernel(dst_hbm, ones_hbm, out_hbm, idx_v, ones_v, acc_sh, sem):
    c = lax.axis_index("c")
    s = lax.axis_index("s")
    wid = c * 16 + s
    pltpu.sync_copy(ones_hbm.at[pl.ds(0, K)], ones_v)
    pltpu.sync_copy(dst_hbm.at[wid], idx_v)
    pltpu.sync_copy(ones_hbm.at[pl.ds(s * ROWS, ROWS)],
                    acc_sh.at[pl.ds(s * ROWS, ROWS)])
    plsc.subcore_barrier()

    def fire(j, carry):
        pltpu.async_copy(ones_v, acc_sh.at[idx_v.at[j]], sem, add=True)
        return carry

    lax.fori_loop(0, NCH, fire, 0)

    def drain(j, carry):
        pltpu.make_async_copy(ones_v, acc_sh.at[idx_v.at[0]], sem).wait()
        return carry

    lax.fori_loop(0, NCH, drain, 0)
    plsc.subcore_barrier()
    pltpu.sync_copy(acc_sh.at[pl.ds(s * ROWS, ROWS)],
                    out_hbm.at[c, pl.ds(s * ROWS, ROWS)])


@functools.partial(
    pl.kernel,
    out_type=jax.ShapeDtypeStruct((2, NPAD, C), jnp.float32),
    mesh=_mesh,
    scratch_types=[
        pltpu.VMEM((NCH, K), jnp.int32),
        pltpu.VMEM((2, 2, K), jnp.int32),
        pltpu.VMEM((2, K, C), jnp.float32),
        pltpu.VMEM_SHARED((NPAD, C), jnp.float32),
        pltpu.SemaphoreType.DMA,
        pltpu.SemaphoreType.DMA,
    ],
)
def _agg_kernel(hwn_hbm, eidx_hbm, out_hbm,
                pidx_v, idx_ring, rows_v, acc_sh, gsem, ssem):
    c = lax.axis_index("c")
    s = lax.axis_index("s")
    wid = c * 16 + s
    pltpu.sync_copy(eidx_hbm.at[wid], pidx_v)
    pltpu.sync_copy(hwn_hbm.at[pl.ds(s * ROWS, ROWS)],
                    acc_sh.at[pl.ds(s * ROWS, ROWS)])

    def unpack(j, slot):
        for k in range(K // 16):
            pv = pidx_v[j, pl.ds(k * 16, 16)]
            idx_ring[slot, 0, pl.ds(k * 16, 16)] = lax.shift_right_logical(
                pv, 16)
            idx_ring[slot, 1, pl.ds(k * 16, 16)] = lax.bitwise_and(
                pv, 0xFFFF)

    unpack(0, 0)
    pltpu.async_copy(hwn_hbm.at[idx_ring.at[0, 0]], rows_v.at[0], gsem)
    plsc.subcore_barrier()

    def body(j, carry):
        b = lax.rem(j, 2)
        bn = 1 - b
        pltpu.make_async_copy(hwn_hbm.at[idx_ring.at[b, 0]],
                              rows_v.at[b], gsem).wait()

        @pl.when(j > 0)
        def _():
            pltpu.make_async_copy(rows_v.at[bn],
                                  acc_sh.at[idx_ring.at[bn, 1]], ssem).wait()

        pltpu.async_copy(rows_v.at[b], acc_sh.at[idx_ring.at[b, 1]],
                         ssem, add=True)

        @pl.when(j + 1 < NCH)
        def _():
            unpack(j + 1, bn)
            pltpu.async_copy(hwn_hbm.at[idx_ring.at[bn, 0]],
                             rows_v.at[bn], gsem)

        return carry

    lax.fori_loop(0, NCH, body, 0)
    pltpu.make_async_copy(rows_v.at[lax.rem(NCH - 1, 2)],
                          acc_sh.at[idx_ring.at[lax.rem(NCH - 1, 2), 1]],
                          ssem).wait()
    plsc.subcore_barrier()
    pltpu.sync_copy(acc_sh.at[pl.ds(s * ROWS, ROWS)],
                    out_hbm.at[c, pl.ds(s * ROWS, ROWS)])


BLK = 1024
GRID = NPAD // BLK


def _agg_to_bn(acc_ref, hwn_ref, deg_ref, gcnb_ref):
    accblk = acc_ref[0] + acc_ref[1] - hwn_ref[...]
    dinv = lax.rsqrt(deg_ref[...])
    return accblk * dinv + gcnb_ref[...]


def _postbn_body(acc_ref, hwn_ref, deg_ref, gcnb_ref, bng_ref, bnb_ref,
                 wn_ref, h_out, hwnn_out, sum_ref, sq_ref):
    p = pl.program_id(0)
    i = pl.program_id(1)
    agg = _agg_to_bn(acc_ref, hwn_ref, deg_ref, gcnb_ref)

    @pl.when(p == 0)
    def _():
        rows = i * BLK + lax.broadcasted_iota(jnp.int32, (BLK, 1), 0)
        m = jnp.where(rows < N, agg, 0.0)
        ssum = jnp.sum(m, axis=0, keepdims=True)
        ssq = jnp.sum(m * m, axis=0, keepdims=True)

        @pl.when(i == 0)
        def _():
            sum_ref[...] = ssum
            sq_ref[...] = ssq

        @pl.when(i > 0)
        def _():
            sum_ref[...] += ssum
            sq_ref[...] += ssq

    @pl.when(p == 1)
    def _():
        inv_n = jnp.float32(1.0 / N)
        mean = sum_ref[...] * inv_n
        var = sq_ref[...] * inv_n - mean * mean
        rstd = lax.rsqrt(var + 1e-5)
        h2 = jnp.maximum((agg - mean) * rstd * bng_ref[...] + bnb_ref[...],
                         0.0)
        h_out[...] = h2
        dinv = lax.rsqrt(deg_ref[...])
        hwnn_out[...] = dinv * jnp.dot(h2, wn_ref[...],
                                       preferred_element_type=jnp.float32)


def _postbn_call(acc, hwn, deg, gcnb, bng, bnb, wnext):
    vec = pl.BlockSpec((1, C), lambda p, i: (0, 0))
    ospec = pl.BlockSpec((BLK, C), lambda p, i: (i * p, 0))
    return pl.pallas_call(
        _postbn_body,
        grid=(2, GRID),
        in_specs=[
            pl.BlockSpec((2, BLK, C), lambda p, i: (0, i, 0)),
            pl.BlockSpec((BLK, C), lambda p, i: (i, 0)),
            pl.BlockSpec((BLK, 1), lambda p, i: (i, 0)),
            vec, vec, vec,
            pl.BlockSpec((C, C), lambda p, i: (0, 0)),
        ],
        out_specs=[ospec, ospec],
        out_shape=[
            jax.ShapeDtypeStruct((NPAD, C), jnp.float32),
            jax.ShapeDtypeStruct((NPAD, C), jnp.float32),
        ],
        scratch_shapes=[
            pltpu.VMEM((1, C), jnp.float32),
            pltpu.VMEM((1, C), jnp.float32),
        ],
    )(acc, hwn, deg, gcnb, bng, bnb, wnext)


MBLK = 400
MGRID = N // MBLK


def _mlp_body(out_in_ref, h_ref, w1_ref, b1_ref, lng_ref, lnb_ref,
              w2_ref, b2_ref, t_out):
    tt = jnp.dot(h_ref[...], w1_ref[...], preferred_element_type=jnp.float32)
    tt = tt + b1_ref[...]
    mu = jnp.mean(tt, axis=1, keepdims=True)
    vv = jnp.mean(tt * tt, axis=1, keepdims=True) - mu * mu
    tt = (tt - mu) * lax.rsqrt(vv + 1e-5) * lng_ref[...] + lnb_ref[...]
    tt = jnp.maximum(tt, 0.0)
    t_out[...] = jnp.dot(tt, w2_ref[...],
                         preferred_element_type=jnp.float32)[None] + b2_ref[...]


def _mlp_call(i, out_buf, h2, w1, b1, lng, lnb, w2, b2):
    vec = pl.BlockSpec((1, C), lambda n: (0, 0))
    mat = pl.BlockSpec((C, C), lambda n: (0, 0))
    return pl.pallas_call(
        _mlp_body,
        grid=(MGRID,),
        in_specs=[
            pl.BlockSpec((1, MBLK, C), lambda n: (0, n, i)),
            pl.BlockSpec((MBLK, C), lambda n: (n, 0)),
            mat, vec, vec, vec, mat, vec,
        ],
        out_specs=pl.BlockSpec((1, MBLK, C), lambda n: (0, n, i)),
        out_shape=jax.ShapeDtypeStruct((1, N, L * C), jnp.float32),
        input_output_aliases={0: 0},
    )(out_buf, h2, w1, b1, lng, lnb, w2, b2)


def _mm_body(h_ref, w_ref, out_ref):
    out_ref[...] = jnp.dot(h_ref[...], w_ref[...],
                           preferred_element_type=jnp.float32)


def _mm_call(h, w):
    return pl.pallas_call(
        _mm_body,
        grid=(GRID,),
        in_specs=[
            pl.BlockSpec((BLK, C), lambda i: (i, 0)),
            pl.BlockSpec((C, C), lambda i: (0, 0)),
        ],
        out_specs=pl.BlockSpec((BLK, C), lambda i: (i, 0)),
        out_shape=jax.ShapeDtypeStruct((NPAD, C), jnp.float32),
    )(h, w)


def _scale_body(xw_ref, parts_ref, out_ref):
    deg = parts_ref[0, :, 0:1] + parts_ref[1, :, 0:1] - 1.0
    out_ref[...] = xw_ref[...] * lax.rsqrt(deg)


def _scale_call(xw, deg_parts):
    return pl.pallas_call(
        _scale_body,
        grid=(GRID,),
        in_specs=[
            pl.BlockSpec((BLK, C), lambda i: (i, 0)),
            pl.BlockSpec((2, BLK, 16), lambda i: (0, i, 0)),
        ],
        out_specs=pl.BlockSpec((BLK, C), lambda i: (i, 0)),
        out_shape=jax.ShapeDtypeStruct((NPAD, C), jnp.float32),
    )(xw, deg_parts)


def kernel(x, edge_index, batch, gcn_W, gcn_b, bn_g, bn_b,
           mlp_W1, mlp_b1, ln_g, ln_b, mlp_W2, mlp_b2):
    src = edge_index[0]
    dst = edge_index[1]
    pad = EPAD - E
    trash = N + (jnp.arange(pad, dtype=jnp.int32) % (NPAD - N))
    src_p = jnp.concatenate([src, trash])
    dst_p = jnp.concatenate([dst, trash])
    packed = jnp.left_shift(src_p, 16) + dst_p
    eidx = packed.reshape(32, NCH, K)
    dst_d = dst_p.reshape(32, NCH, K)
    ones_nd = jnp.ones((NPAD, 16), jnp.float32)

    x_pad = jnp.zeros((NPAD, C), jnp.float32).at[:N].set(x)

    deg_parts = _deg_kernel(dst_d, ones_nd)
    xw = _mm_call(x_pad, gcn_W[0])
    hwn = _scale_call(xw, deg_parts)
    deg = (deg_parts[0, :, 0] + deg_parts[1, :, 0] - 1.0).reshape(NPAD, 1)

    r = lambda a: a.reshape(1, C)
    out = jnp.zeros((1, N, L * C), jnp.float32)
    for i in range(L):
        acc = _agg_kernel(hwn, eidx)
        h, hwn = _postbn_call(acc, hwn, deg, r(gcn_b[i]),
                              r(bn_g[i]), r(bn_b[i]), gcn_W[(i + 1) % L])
        out = _mlp_call(i, out, h, mlp_W1[i], r(mlp_b1[i]), r(ln_g[i]),
                        r(ln_b[i]), mlp_W2[i], r(mlp_b2[i]))
    return out
